# baseline (speedup 1.0000x reference)
import functools

import jax
import jax.numpy as jnp
from jax import lax
from jax.experimental import pallas as pl
from jax.experimental.pallas import tpu as pltpu
from jax.experimental.pallas import tpu_sc as plsc

_VOCAB = 1000
_D = 300
_K = 64
_B = 4096
_L = 50
_O = 14

_NWORD = _K // 2
_WSTRIDE = _NWORD + 1

_NW = 32
_BPW = _B // _NW
_NG = _BPW // 16


def _proj_body(table_ref, w1_ref, gamma_ref, rv_ref, t2_ref):
    inv = gamma_ref[...] * lax.rsqrt(rv_ref[...] + 1e-5)
    mm = lax.dot_general(
        table_ref[...], w1_ref[...],
        dimension_numbers=(((1,), (1,)), ((), ())),
        preferred_element_type=jnp.float32)
    t2_ref[...] = (mm * (inv * (1.0 / _L))).astype(jnp.bfloat16)


def _sc_pool_body(xT_hbm, t2_hbm, out_hbm, x_v, t2_v, acc_v, sem):
    wid = lax.axis_index("s") * 2 + lax.axis_index("c")
    base = wid * _BPW

    t2_cp = pltpu.async_copy(t2_hbm, t2_v, sem)
    pltpu.sync_copy(xT_hbm.at[:, pl.ds(base, _BPW)], x_v)
    t2_cp.wait()

    zeros16 = jnp.zeros((16,), jnp.float32)

    def g_body(g, carry):
        gbase = g * 16

        for wc in range(4):
            def j_body(j, accs):
                idx = x_v[j, pl.ds(gbase, 16)] * _WSTRIDE + (wc * 8)
                new = list(accs)
                for w in range(8):
                    word = plsc.load_gather(t2_v, [idx + w])
                    lo, hi = plsc.unpack(
                        plsc.bitcast(word, jnp.bfloat16),
                        format=plsc.PackFormat.INTERLEAVED)
                    new[2 * w] = new[2 * w] + lo
                    new[2 * w + 1] = new[2 * w + 1] + hi
                return tuple(new)

            accs = plsc.parallel_loop(0, _L, unroll=2,
                                      carry=(zeros16,) * 16)(j_body)
            for i in range(16):
                acc_v[wc * 16 + i, pl.ds(gbase, 16)] = accs[i]
        return carry

    lax.fori_loop(0, _NG, g_body, 0)

    pltpu.sync_copy(acc_v, out_hbm.at[:, pl.ds(base, _BPW)])


_sc_pool = functools.partial(
    pl.kernel,
    out_type=jax.ShapeDtypeStruct((_K, _B), jnp.float32),
    mesh=plsc.VectorSubcoreMesh(core_axis_name="c", subcore_axis_name="s"),
    compiler_params=pltpu.CompilerParams(needs_layout_passes=False),
    scratch_types=[
        pltpu.VMEM((_L, _BPW), jnp.int32),
        pltpu.VMEM((_VOCAB * _WSTRIDE,), jnp.int32),
        pltpu.VMEM((_K, _BPW), jnp.float32),
        pltpu.SemaphoreType.DMA,
    ],
)(_sc_pool_body)


def _head_body(pooledT_ref, w2_ref, b1_ref, gamma_ref, beta_ref,
               rm_ref, rv_ref, b2_ref, out_ref):
    inv = gamma_ref[...] * lax.rsqrt(rv_ref[...] + 1e-5)
    c1 = b1_ref[...] * inv + beta_ref[...] - rm_ref[...] * inv
    h = pooledT_ref[...] + c1
    h = jnp.maximum(h, 0.0)
    out = lax.dot_general(
        w2_ref[...], h,
        dimension_numbers=(((1,), (0,)), ((), ())),
        preferred_element_type=jnp.float32)
    out_ref[...] = out + b2_ref[...]


def kernel(x, table, W1, b1, gamma, beta, running_mean, running_var, W2, b2):
    xT = jnp.transpose(x)
    t2 = pl.pallas_call(
        _proj_body,
        out_shape=jax.ShapeDtypeStruct((_VOCAB, _K), jnp.bfloat16),
    )(table, W1, gamma.reshape(1, _K), running_var.reshape(1, _K))

    t2_words = lax.bitcast_convert_type(
        t2.reshape(_VOCAB, _NWORD, 2), jnp.int32)
    t2_words = jnp.pad(t2_words, ((0, 0), (0, _WSTRIDE - _NWORD)))
    pooledT = _sc_pool(xT, t2_words.reshape(_VOCAB * _WSTRIDE))

    outT = pl.pallas_call(
        _head_body,
        out_shape=jax.ShapeDtypeStruct((_O, _B), jnp.float32),
    )(pooledT, W2,
      b1.reshape(_K, 1), gamma.reshape(_K, 1), beta.reshape(_K, 1),
      running_mean.reshape(_K, 1), running_var.reshape(_K, 1),
      b2.reshape(_O, 1))
    return jnp.transpose(outT)


# speedup vs baseline: 1.0312x; 1.0312x over previous
import functools

import jax
import jax.numpy as jnp
from jax import lax
from jax.experimental import pallas as pl
from jax.experimental.pallas import tpu as pltpu
from jax.experimental.pallas import tpu_sc as plsc

_VOCAB = 1000
_D = 300
_K = 64
_B = 4096
_L = 50
_O = 14

_NWORD = _K // 2
_WSTRIDE = _NWORD + 1

_NW = 32
_BPW = _B // _NW
_NG = _BPW // 16


def _proj_body(table_ref, w1_ref, gamma_ref, rv_ref, t2_ref):
    inv = gamma_ref[...] * lax.rsqrt(rv_ref[...] + 1e-5)
    mm = lax.dot_general(
        table_ref[...], w1_ref[...],
        dimension_numbers=(((1,), (1,)), ((), ())),
        preferred_element_type=jnp.float32)
    t2_ref[...] = (mm * (inv * (1.0 / _L))).astype(jnp.bfloat16)


def _sc_pool_body(xT_hbm, t2_hbm, out_hbm, x_v, t2_v, acc_v, sem):
    wid = lax.axis_index("s") * 2 + lax.axis_index("c")
    base = wid * _BPW

    t2_cp = pltpu.async_copy(t2_hbm, t2_v, sem)
    pltpu.sync_copy(xT_hbm.at[:, pl.ds(base, _BPW)], x_v)
    t2_cp.wait()

    zeros16 = jnp.zeros((16,), jnp.float32)

    def g_body(g, carry):
        gbase = g * 16

        for wc in range(2):
            def j_body(j, accs):
                idx = x_v[j, pl.ds(gbase, 16)] * _WSTRIDE + (wc * 16)
                new = list(accs)
                for w in range(16):
                    word = plsc.load_gather(t2_v, [idx + w])
                    lo, hi = plsc.unpack(
                        plsc.bitcast(word, jnp.bfloat16),
                        format=plsc.PackFormat.INTERLEAVED)
                    new[2 * w] = new[2 * w] + lo
                    new[2 * w + 1] = new[2 * w + 1] + hi
                return tuple(new)

            accs = lax.fori_loop(0, _L, j_body, (zeros16,) * 32)
            for i in range(32):
                acc_v[wc * 32 + i, pl.ds(gbase, 16)] = accs[i]
        return carry

    lax.fori_loop(0, _NG, g_body, 0)

    pltpu.sync_copy(acc_v, out_hbm.at[:, pl.ds(base, _BPW)])


_sc_pool = functools.partial(
    pl.kernel,
    out_type=jax.ShapeDtypeStruct((_K, _B), jnp.float32),
    mesh=plsc.VectorSubcoreMesh(core_axis_name="c", subcore_axis_name="s"),
    compiler_params=pltpu.CompilerParams(needs_layout_passes=False),
    scratch_types=[
        pltpu.VMEM((_L, _BPW), jnp.int32),
        pltpu.VMEM((_VOCAB * _WSTRIDE,), jnp.int32),
        pltpu.VMEM((_K, _BPW), jnp.float32),
        pltpu.SemaphoreType.DMA,
    ],
)(_sc_pool_body)


def _head_body(pooledT_ref, w2_ref, b1_ref, gamma_ref, beta_ref,
               rm_ref, rv_ref, b2_ref, out_ref):
    inv = gamma_ref[...] * lax.rsqrt(rv_ref[...] + 1e-5)
    c1 = b1_ref[...] * inv + beta_ref[...] - rm_ref[...] * inv
    h = pooledT_ref[...] + c1
    h = jnp.maximum(h, 0.0)
    out = lax.dot_general(
        w2_ref[...], h,
        dimension_numbers=(((1,), (0,)), ((), ())),
        preferred_element_type=jnp.float32)
    out_ref[...] = out + b2_ref[...]


def kernel(x, table, W1, b1, gamma, beta, running_mean, running_var, W2, b2):
    xT = jnp.transpose(x)
    t2 = pl.pallas_call(
        _proj_body,
        out_shape=jax.ShapeDtypeStruct((_VOCAB, _K), jnp.bfloat16),
    )(table, W1, gamma.reshape(1, _K), running_var.reshape(1, _K))

    t2_words = lax.bitcast_convert_type(
        t2.reshape(_VOCAB, _NWORD, 2), jnp.int32)
    t2_words = jnp.pad(t2_words, ((0, 0), (0, _WSTRIDE - _NWORD)))
    pooledT = _sc_pool(xT, t2_words.reshape(_VOCAB * _WSTRIDE))

    outT = pl.pallas_call(
        _head_body,
        out_shape=jax.ShapeDtypeStruct((_O, _B), jnp.float32),
    )(pooledT, W2,
      b1.reshape(_K, 1), gamma.reshape(_K, 1), beta.reshape(_K, 1),
      running_mean.reshape(_K, 1), running_var.reshape(_K, 1),
      b2.reshape(_O, 1))
    return jnp.transpose(outT)
